# baseline (speedup 1.0000x reference)
import jax
import jax.numpy as jnp
from jax import lax
from jax.experimental import pallas as pl
from jax.experimental.pallas import tpu as pltpu
from jax.experimental.pallas import tpu_sc as plsc

NC = 2
NS = 16
NW = NC * NS
LANES = 16
BATCH = 16384
RANK = 32
BPW = BATCH // NW
CHUNK = 128
NCHUNK = BPW // CHUNK


def _mf_body(uidx_hbm, sidx_hbm, utab_hbm, itab_hbm, out_hbm,
             idx_u, idx_s, rows_u, rows_s, out_v, sem):
  wid = lax.axis_index("s") * NC + lax.axis_index("c")
  base_rows = wid * NCHUNK

  pltpu.sync_copy(uidx_hbm.at[pl.ds(base_rows, NCHUNK)], idx_u)
  pltpu.sync_copy(sidx_hbm.at[pl.ds(base_rows, NCHUNK)], idx_s)

  copies = []
  for j in range(NCHUNK):
    copies.append(pltpu.async_copy(
        utab_hbm.at[idx_u.at[j]], rows_u.at[pl.ds(j * CHUNK, CHUNK)], sem))
    copies.append(pltpu.async_copy(
        itab_hbm.at[idx_s.at[j]], rows_s.at[pl.ds(j * CHUNK, CHUNK)], sem))
  for c in copies:
    c.wait()

  lane = lax.broadcasted_iota(jnp.int32, (LANES,), 0)

  @plsc.parallel_loop(0, BPW // LANES, unroll=1)
  def _(g):
    acc = jnp.zeros((LANES,), jnp.float32)
    for j in range(LANES):
      r = g * LANES + j
      t = (rows_u[r, pl.ds(0, LANES)] * rows_s[r, pl.ds(0, LANES)]
           + rows_u[r, pl.ds(LANES, LANES)] * rows_s[r, pl.ds(LANES, LANES)])
      acc = jnp.where(lane == j, jnp.sum(t), acc)
    out_v[pl.ds(g * LANES, LANES)] = acc

  pltpu.sync_copy(out_v, out_hbm.at[pl.ds(wid * BPW, BPW)])


@jax.jit
def kernel(userIdx, servIdx, user_table, item_table):
  uidx = userIdx.astype(jnp.int32).reshape(NW * NCHUNK, CHUNK)
  sidx = servIdx.astype(jnp.int32).reshape(NW * NCHUNK, CHUNK)
  mesh = plsc.VectorSubcoreMesh(core_axis_name="c", subcore_axis_name="s",
                                num_cores=NC, num_subcores=NS)
  f = pl.kernel(
      _mf_body,
      out_type=jax.ShapeDtypeStruct((BATCH,), jnp.float32),
      mesh=mesh,
      scratch_types=[
          pltpu.VMEM((NCHUNK, CHUNK), jnp.int32),
          pltpu.VMEM((NCHUNK, CHUNK), jnp.int32),
          pltpu.VMEM((BPW, RANK), jnp.float32),
          pltpu.VMEM((BPW, RANK), jnp.float32),
          pltpu.VMEM((BPW,), jnp.float32),
          pltpu.SemaphoreType.DMA,
      ],
      compiler_params=pltpu.CompilerParams(
          needs_layout_passes=False, use_tc_tiling_on_sc=False),
  )
  return f(uidx, sidx, user_table, item_table)


# speedup vs baseline: 1.2627x; 1.2627x over previous
import jax
import jax.numpy as jnp
from jax import lax
from jax.experimental import pallas as pl
from jax.experimental.pallas import tpu as pltpu
from jax.experimental.pallas import tpu_sc as plsc

NC = 2
NS = 16
NW = NC * NS
LANES = 16
BATCH = 16384
RANK = 32
BPW = BATCH // NW
QTR = 128
NQ = BPW // QTR


def _mf_body(uidx_hbm, sidx_hbm, utab_hbm, itab_hbm, out_hbm,
             idx_u, idx_s, bufs_u0, bufs_u1, bufs_s0, bufs_s1, out_v,
             sem_u0, sem_u1, sem_s0, sem_s1):
  wid = lax.axis_index("s") * NC + lax.axis_index("c")
  base = wid * BPW

  pltpu.sync_copy(uidx_hbm.at[pl.ds(base, BPW)], idx_u)
  pltpu.sync_copy(sidx_hbm.at[pl.ds(base, BPW)], idx_s)

  bufs_u = [bufs_u0, bufs_u1]
  bufs_s = [bufs_s0, bufs_s1]
  sems_u = [sem_u0, sem_u1]
  sems_s = [sem_s0, sem_s1]

  def issue(q, slot):
    bu, bs = bufs_u[slot], bufs_s[slot]
    su, ss = sems_u[slot], sems_s[slot]

    def issue_group(g, _):
      vu = idx_u[pl.ds(q * QTR + g * LANES, LANES)]
      vs = idx_s[pl.ds(q * QTR + g * LANES, LANES)]
      for j in range(LANES):
        r = g * LANES + j
        pltpu.make_async_copy(
            utab_hbm.at[pl.ds(vu[j], 1)], bu.at[pl.ds(r, 1)], su).start()
        pltpu.make_async_copy(
            itab_hbm.at[pl.ds(vs[j], 1)], bs.at[pl.ds(r, 1)], ss).start()
      return 0

    lax.fori_loop(0, QTR // LANES, issue_group, 0)

  lane = lax.broadcasted_iota(jnp.int32, (LANES,), 0)

  def compute(q, slot):
    bu, bs = bufs_u[slot], bufs_s[slot]
    pltpu.make_async_copy(utab_hbm.at[pl.ds(0, QTR)], bu, sems_u[slot]).wait()
    pltpu.make_async_copy(itab_hbm.at[pl.ds(0, QTR)], bs, sems_s[slot]).wait()

    @plsc.parallel_loop(0, QTR // LANES, unroll=1)
    def _(g):
      acc = jnp.zeros((LANES,), jnp.float32)
      for j in range(LANES):
        r = g * LANES + j
        t = (bu[r, pl.ds(0, LANES)] * bs[r, pl.ds(0, LANES)]
             + bu[r, pl.ds(LANES, LANES)] * bs[r, pl.ds(LANES, LANES)])
        acc = jnp.where(lane == j, jnp.sum(t), acc)
      out_v[pl.ds(q * QTR + g * LANES, LANES)] = acc

  issue(0, 0)
  for q in range(NQ):
    if q + 1 < NQ:
      issue(q + 1, (q + 1) % 2)
    compute(q, q % 2)

  pltpu.sync_copy(out_v, out_hbm.at[pl.ds(base, BPW)])


@jax.jit
def kernel(userIdx, servIdx, user_table, item_table):
  uidx = userIdx.astype(jnp.int32)
  sidx = servIdx.astype(jnp.int32)
  mesh = plsc.VectorSubcoreMesh(core_axis_name="c", subcore_axis_name="s",
                                num_cores=NC, num_subcores=NS)
  f = pl.kernel(
      _mf_body,
      out_type=jax.ShapeDtypeStruct((BATCH,), jnp.float32),
      mesh=mesh,
      scratch_types=[
          pltpu.VMEM((BPW,), jnp.int32),
          pltpu.VMEM((BPW,), jnp.int32),
          pltpu.VMEM((QTR, RANK), jnp.float32),
          pltpu.VMEM((QTR, RANK), jnp.float32),
          pltpu.VMEM((QTR, RANK), jnp.float32),
          pltpu.VMEM((QTR, RANK), jnp.float32),
          pltpu.VMEM((BPW,), jnp.float32),
          pltpu.SemaphoreType.DMA,
          pltpu.SemaphoreType.DMA,
          pltpu.SemaphoreType.DMA,
          pltpu.SemaphoreType.DMA,
      ],
      compiler_params=pltpu.CompilerParams(needs_layout_passes=False),
  )
  return f(uidx, sidx, user_table, item_table)
